# Initial kernel scaffold; baseline (speedup 1.0000x reference)
import jax
import jax.numpy as jnp
from jax.experimental import pallas as pl


def kernel(user_emb, item_emb, edge_index, user_intent, item_intent):
    raise NotImplementedError("write your pallas kernel here")


# speedup vs baseline: 9.8557x; 9.8557x over previous
import functools

import jax
import jax.numpy as jnp
from jax import lax
from jax.experimental import pallas as pl
from jax.experimental.pallas import tpu as pltpu
from jax.experimental.pallas import tpu_sc as plsc

NU = 5000
NI = 5000
NN = NU + NI
D = 128
NP = 10240
NE = 160000
E2 = 2 * NE
NTILES = 32
CHUNK = 128
NCH = (E2 + NTILES * CHUNK - 1) // (NTILES * CHUNK)
EPAD = NTILES * CHUNK * NCH
EPW = EPAD // NTILES
RPT = NP // 16

_mesh = plsc.VectorSubcoreMesh(core_axis_name="c", subcore_axis_name="s")


def _deg_body(dst_hbm, out_hbm, idx_v, deg_v, sem):
    cid = lax.axis_index("c")
    sid = lax.axis_index("s")
    wid = sid * 2 + cid

    zeros16 = jnp.zeros((16,), jnp.float32)

    def zero_body(i, _):
        deg_v[pl.ds(i * 16, 16)] = zeros16
        return ()
    lax.fori_loop(0, NP // 16, zero_body, ())

    pltpu.sync_copy(dst_hbm.at[wid], idx_v)

    ones16 = jnp.ones((16,), jnp.float32)

    def body(k, _):
        idx16 = idx_v[pl.ds(k * 16, 16)]
        plsc.addupdate_scatter(deg_v, [idx16], ones16)
        return ()
    lax.fori_loop(0, EPW // 16, body, ())

    pltpu.sync_copy(deg_v, out_hbm.at[wid])


_deg_kernel = functools.partial(
    pl.kernel,
    out_type=jax.ShapeDtypeStruct((NTILES, NP), jnp.float32),
    mesh=_mesh,
    compiler_params=pltpu.CompilerParams(needs_layout_passes=False),
    scratch_types=[
        pltpu.VMEM((EPW,), jnp.int32),
        pltpu.VMEM((NP,), jnp.float32),
        pltpu.SemaphoreType.DMA,
    ],
)(_deg_body)


def _spmm_body(y_hbm, src_hbm, dst_hbm, out_hbm, srcv, dstv, rows, acc_sh, sem):
    cid = lax.axis_index("c")
    sid = lax.axis_index("s")
    wid = sid * 2 + cid

    zeros16 = jnp.zeros((16,), jnp.float32)

    def zero_body(k, _):
        r = k // (D // 16)
        c = k % (D // 16)
        rows[r, pl.ds(c * 16, 16)] = zeros16
        return ()
    lax.fori_loop(0, CHUNK * (D // 16), zero_body, ())

    def zcopy(b, _):
        pltpu.sync_copy(rows, acc_sh.at[pl.ds(sid * RPT + b * CHUNK, CHUNK)])
        return ()
    lax.fori_loop(0, RPT // CHUNK, zcopy, ())
    plsc.subcore_barrier()

    pltpu.sync_copy(src_hbm.at[wid], srcv)
    pltpu.sync_copy(dst_hbm.at[wid], dstv)

    def body(j, _):
        pltpu.async_copy(y_hbm.at[srcv.at[j]], rows, sem).wait()
        pltpu.sync_copy(rows, acc_sh.at[dstv.at[j]], add=True)
        return ()
    lax.fori_loop(0, NCH, body, ())

    plsc.subcore_barrier()
    pltpu.sync_copy(acc_sh.at[pl.ds(sid * RPT, RPT)],
                    out_hbm.at[cid, pl.ds(sid * RPT, RPT)])


_spmm_kernel = functools.partial(
    pl.kernel,
    out_type=jax.ShapeDtypeStruct((2, NP, D), jnp.float32),
    mesh=_mesh,
    scratch_types=[
        pltpu.VMEM((NCH, CHUNK), jnp.int32),
        pltpu.VMEM((NCH, CHUNK), jnp.int32),
        pltpu.VMEM((CHUNK, D), jnp.float32),
        pltpu.VMEM_SHARED((NP, D), jnp.float32),
        pltpu.SemaphoreType.DMA,
    ],
)(_spmm_body)


BLK = 1000


def _tc_layer_body(x_ref, a0_ref, a1_ref, db_ref, wu_ref, wi_ref,
                   msg_ref, int_ref, xn_ref, yn_ref):
    i = pl.program_id(0)
    x = x_ref[...]
    db = db_ref[...]
    msg = (a0_ref[...] + a1_ref[...]) * db
    w = jnp.where(i < NU // BLK, wu_ref[...], wi_ref[...])
    logits = jnp.dot(x, w, preferred_element_type=jnp.float32)
    m = jnp.max(logits, axis=1, keepdims=True)
    e = jnp.exp(logits - m)
    p = e / jnp.sum(e, axis=1, keepdims=True)
    itl = lax.dot_general(p, w, (((1,), (1,)), ((), ())),
                          preferred_element_type=jnp.float32)
    msg_ref[...] = msg
    int_ref[...] = itl
    xn = msg + itl + x
    xn_ref[...] = xn
    yn_ref[...] = xn * db


def _tc_layer(x, a0, a1, disb, wu, wi):
    grid = (NN // BLK,)
    row_spec = pl.BlockSpec((BLK, D), lambda i: (i, 0))
    w_spec = pl.BlockSpec((D, D), lambda i: (0, 0))
    out_sds = jax.ShapeDtypeStruct((NN, D), jnp.float32)
    return pl.pallas_call(
        _tc_layer_body,
        grid=grid,
        in_specs=[row_spec, row_spec, row_spec, row_spec, w_spec, w_spec],
        out_specs=[row_spec, row_spec, row_spec, row_spec],
        out_shape=[out_sds, out_sds, out_sds, out_sds],
    )(x, a0, a1, disb, wu, wi)


def kernel(user_emb, item_emb, edge_index, user_intent, item_intent):
    h = edge_index[0].astype(jnp.int32)
    t = edge_index[1].astype(jnp.int32) + NU
    all_h = jnp.concatenate([h, t])
    all_t = jnp.concatenate([t, h])
    npad = EPAD - E2
    src = jnp.concatenate([all_t, jnp.zeros((npad,), jnp.int32)])
    dst = jnp.concatenate([all_h, jnp.full((npad,), NN, jnp.int32)])
    src3 = src.reshape(NTILES, NCH, CHUNK)
    dst3 = dst.reshape(NTILES, NCH, CHUNK)
    dst2 = dst.reshape(NTILES, EPW)

    degp = _deg_kernel(dst2)
    deg = jnp.sum(degp, axis=0)[:NN]
    dis = jnp.where(deg > 0, lax.rsqrt(jnp.maximum(deg, 1.0)), 0.0)
    disb = jnp.broadcast_to(dis[:, None], (NN, D))

    e0 = jnp.concatenate([user_emb, item_emb], axis=0)
    y0 = e0 * disb

    acc0 = _spmm_kernel(y0, src3, dst3)
    msg0, int0, e1, y1 = _tc_layer(e0, acc0[0, :NN], acc0[1, :NN], disb,
                                   user_intent, item_intent)

    acc1 = _spmm_kernel(y1, src3, dst3)
    msg1, int1, e2, _ = _tc_layer(e1, acc1[0, :NN], acc1[1, :NN], disb,
                                  user_intent, item_intent)

    final = e0 + e1 + e2
    return (final[:NU], final[NU:],
            jnp.stack([msg0, msg1], axis=0),
            jnp.stack([int0, int1], axis=0))
